# baseline (speedup 1.0000x reference)
import functools

import jax
import jax.numpy as jnp
from jax import lax
from jax.experimental import pallas as pl
from jax.experimental.pallas import tpu as pltpu
from jax.experimental.pallas import tpu_sc as plsc

_EPS = 1e-8
_CA = 512
_RP = 832
_SC_ITEMS = 1024
_PW = _SC_ITEMS // 32
_NEG_BIG = -3.0e38


def _argmax_body(a_s_ref, a_e_ref, b_s_ref, b_e_ref, rowbase_ref, cls_ref,
                 bestv_ref, besti_ref, sidx_ref, bidx0_ref, num_classes):
    i = pl.program_id(0)

    @pl.when(i == 0)
    def _init():
        bestv_ref[...] = jnp.full(bestv_ref.shape, _NEG_BIG, jnp.float32)
        besti_ref[...] = jnp.zeros(besti_ref.shape, jnp.int32)

    a_s = a_s_ref[...]
    a_e = a_e_ref[...]
    b_s = b_s_ref[...]
    b_e = b_e_ref[...]
    inter = jnp.maximum(jnp.minimum(a_e, b_e) - jnp.maximum(a_s, b_s), 0.0)
    denom = (a_e - a_s) + (b_e - b_s + _EPS) - inter
    iou = inter / denom
    mx = jnp.max(iou, axis=1, keepdims=True)
    iota = lax.broadcasted_iota(jnp.int32, iou.shape, 1)
    lid = jnp.min(jnp.where(iou == mx, iota, jnp.int32(2 ** 30)),
                  axis=1, keepdims=True) + i * _CA
    upd = mx > bestv_ref[...]
    besti_ref[...] = jnp.where(upd, lid, besti_ref[...])
    bestv_ref[...] = jnp.where(upd, mx, bestv_ref[...])

    @pl.when(i == pl.num_programs(0) - 1)
    def _emit():
        r = rowbase_ref[...] + besti_ref[...]
        sidx_ref[...] = r * num_classes + cls_ref[...]
        bidx0_ref[...] = r * 2


def _best_anchor_indices(anchors, gt_bboxes, gt_classes):
    b, g = gt_classes.shape
    a = anchors.shape[0]
    a_pad = ((a + _CA - 1) // _CA) * _CA
    rows = b * g
    pad_a = a_pad - a
    a_s = jnp.pad(anchors[:, 0], (0, pad_a), constant_values=2.0).reshape(1, a_pad)
    a_e = jnp.pad(anchors[:, 1], (0, pad_a), constant_values=2.0).reshape(1, a_pad)
    pad_r = _RP - rows
    b_s = jnp.pad(gt_bboxes[..., 0].reshape(-1), (0, pad_r),
                  constant_values=1.0).reshape(_RP, 1)
    b_e = jnp.pad(gt_bboxes[..., 1].reshape(-1), (0, pad_r),
                  constant_values=0.0).reshape(_RP, 1)
    rowbase = jnp.pad(jnp.repeat(jnp.arange(b, dtype=jnp.int32) * a, g),
                      (0, pad_r)).reshape(_RP, 1)
    cls = jnp.pad(gt_classes.astype(jnp.int32).reshape(-1),
                  (0, pad_r)).reshape(_RP, 1)

    grid = a_pad // _CA
    row_spec = pl.BlockSpec((_RP, 1), lambda i: (0, 0))
    outs = pl.pallas_call(
        functools.partial(_argmax_body, num_classes=20),
        grid=(grid,),
        in_specs=[
            pl.BlockSpec((1, _CA), lambda i: (0, i)),
            pl.BlockSpec((1, _CA), lambda i: (0, i)),
            row_spec, row_spec, row_spec, row_spec,
        ],
        out_specs=[row_spec, row_spec, row_spec, row_spec],
        out_shape=[
            jax.ShapeDtypeStruct((_RP, 1), jnp.float32),
            jax.ShapeDtypeStruct((_RP, 1), jnp.int32),
            jax.ShapeDtypeStruct((_RP, 1), jnp.int32),
            jax.ShapeDtypeStruct((_RP, 1), jnp.int32),
        ],
    )(a_s, a_e, b_s, b_e, rowbase, cls)
    return outs[2].reshape(-1), outs[3].reshape(-1)


def _softplus_body(x_ref, out_ref):
    i = pl.program_id(0)

    @pl.when(i == 0)
    def _init():
        out_ref[0, 0] = 0.0

    x = x_ref[...]
    out_ref[0, 0] += jnp.sum(jnp.maximum(x, 0.0)
                             + jnp.log1p(jnp.exp(-jnp.abs(x))))


def _softplus_sum(scores):
    n = scores.size
    x2d = scores.reshape(n // 128, 128)
    steps = 25
    blk = x2d.shape[0] // steps
    out = pl.pallas_call(
        _softplus_body,
        grid=(steps,),
        in_specs=[pl.BlockSpec((blk, 128), lambda i: (i, 0))],
        out_specs=pl.BlockSpec(memory_space=pltpu.SMEM),
        out_shape=jax.ShapeDtypeStruct((1, 1), jnp.float32),
    )(x2d)
    return out[0, 0]


def _sc_body(sflat, bflat, sidx, bidx0, bidx1, gs, ge, keep, outc, outr,
             sidx_v, b0i_v, b1i_v, gs_v, ge_v, kp_v, sv_v, b0_v, b1_v,
             accc_v, accr_v, sem):
    cid = lax.axis_index("c")
    sid = lax.axis_index("s")
    wid = sid * 2 + cid
    base = wid * _PW
    pltpu.sync_copy(sidx.at[pl.ds(base, _PW)], sidx_v)
    pltpu.sync_copy(bidx0.at[pl.ds(base, _PW)], b0i_v)
    pltpu.sync_copy(bidx1.at[pl.ds(base, _PW)], b1i_v)
    pltpu.sync_copy(gs.at[pl.ds(base, _PW)], gs_v)
    pltpu.sync_copy(ge.at[pl.ds(base, _PW)], ge_v)
    pltpu.sync_copy(keep.at[pl.ds(base, _PW)], kp_v)
    pltpu.async_copy(sflat.at[sidx_v], sv_v, sem).wait()
    pltpu.async_copy(bflat.at[b0i_v], b0_v, sem).wait()
    pltpu.async_copy(bflat.at[b1i_v], b1_v, sem).wait()
    accc = jnp.zeros((16,), jnp.float32)
    accr = jnp.zeros((16,), jnp.float32)
    for j in range(_PW // 16):
        sl = pl.ds(j * 16, 16)
        k = kp_v[sl]
        accc = accc + sv_v[sl] * k
        for bv, gv in ((b0_v[sl], gs_v[sl]), (b1_v[sl], ge_v[sl])):
            pred = jnp.where(gv != 0.0, bv, jnp.zeros((16,), jnp.float32))
            d = pred - gv
            ad = jnp.abs(d)
            sl1 = jnp.where(ad < 1.0, 0.5 * d * d, ad - 0.5)
            accr = accr + k * sl1
    accc_v[...] = accc
    accr_v[...] = accr
    pltpu.sync_copy(accc_v, outc.at[wid])
    pltpu.sync_copy(accr_v, outr.at[wid])


def _sc_corrections(scores, bboxes, sidx, bidx0, gt_bboxes):
    rows = gt_bboxes.shape[0] * gt_bboxes.shape[1]
    pad = _SC_ITEMS - rows
    sidx_p = jnp.pad(sidx[:rows], (0, pad))
    bidx0_p = jnp.pad(bidx0[:rows], (0, pad))
    bidx1_p = bidx0_p + 1
    gs = jnp.pad(gt_bboxes[..., 0].reshape(-1), (0, pad))
    ge = jnp.pad(gt_bboxes[..., 1].reshape(-1), (0, pad))
    keep = jnp.pad(jnp.ones((rows,), jnp.float32), (0, pad))
    mesh = plsc.VectorSubcoreMesh(core_axis_name="c", subcore_axis_name="s")
    outc, outr = pl.kernel(
        _sc_body,
        out_type=(jax.ShapeDtypeStruct((32, 16), jnp.float32),
                  jax.ShapeDtypeStruct((32, 16), jnp.float32)),
        mesh=mesh,
        scratch_types=[
            pltpu.VMEM((_PW,), jnp.int32),
            pltpu.VMEM((_PW,), jnp.int32),
            pltpu.VMEM((_PW,), jnp.int32),
            pltpu.VMEM((_PW,), jnp.float32),
            pltpu.VMEM((_PW,), jnp.float32),
            pltpu.VMEM((_PW,), jnp.float32),
            pltpu.VMEM((_PW,), jnp.float32),
            pltpu.VMEM((_PW,), jnp.float32),
            pltpu.VMEM((_PW,), jnp.float32),
            pltpu.VMEM((16,), jnp.float32),
            pltpu.VMEM((16,), jnp.float32),
            pltpu.SemaphoreType.DMA,
        ],
    )(scores.reshape(-1), bboxes.reshape(-1), sidx_p, bidx0_p, bidx1_p,
      gs, ge, keep)
    return jnp.sum(outc), jnp.sum(outr)


def kernel(scores, bboxes, gt_classes, gt_bboxes, anchors):
    sidx, bidx0 = _best_anchor_indices(anchors, gt_bboxes, gt_classes)
    s0 = _softplus_sum(scores)
    corr_c, reg = _sc_corrections(scores, bboxes, sidx, bidx0, gt_bboxes)
    return s0 - corr_c + reg


# speedup vs baseline: 2.0243x; 2.0243x over previous
import functools

import jax
import jax.numpy as jnp
from jax import lax
from jax.experimental import pallas as pl
from jax.experimental.pallas import tpu as pltpu
from jax.experimental.pallas import tpu_sc as plsc

_EPS = 1e-8
_CA = 512
_RP = 832
_SC_ITEMS = 1024
_PW = _SC_ITEMS // 32
_NEG_BIG = -3.0e38


def _argmax_body(a_s_ref, a_e_ref, b_s_ref, b_e_ref, rowbase_ref,
                 bestv_ref, besti_ref, bidx0_ref):
    i = pl.program_id(0)

    @pl.when(i == 0)
    def _init():
        bestv_ref[...] = jnp.full(bestv_ref.shape, _NEG_BIG, jnp.float32)
        besti_ref[...] = jnp.zeros(besti_ref.shape, jnp.int32)

    a_s = a_s_ref[...]
    a_e = a_e_ref[...]
    b_s = b_s_ref[...]
    b_e = b_e_ref[...]
    inter = jnp.maximum(jnp.minimum(a_e, b_e) - jnp.maximum(a_s, b_s), 0.0)
    denom = (a_e - a_s) + (b_e - b_s + _EPS) - inter
    iou = inter / denom
    mx = jnp.max(iou, axis=1, keepdims=True)
    iota = lax.broadcasted_iota(jnp.int32, iou.shape, 1)
    lid = jnp.min(jnp.where(iou == mx, iota, jnp.int32(2 ** 30)),
                  axis=1, keepdims=True) + i * _CA
    upd = mx > bestv_ref[...]
    besti_ref[...] = jnp.where(upd, lid, besti_ref[...])
    bestv_ref[...] = jnp.where(upd, mx, bestv_ref[...])

    @pl.when(i == pl.num_programs(0) - 1)
    def _emit():
        bidx0_ref[...] = (rowbase_ref[...] + besti_ref[...]) * 2


def _best_anchor_indices(anchors, gt_bboxes):
    b, g = gt_bboxes.shape[0], gt_bboxes.shape[1]
    a = anchors.shape[0]
    a_pad = ((a + _CA - 1) // _CA) * _CA
    rows = b * g
    pad_a = a_pad - a
    a_s = jnp.pad(anchors[:, 0], (0, pad_a), constant_values=2.0).reshape(1, a_pad)
    a_e = jnp.pad(anchors[:, 1], (0, pad_a), constant_values=2.0).reshape(1, a_pad)
    pad_r = _RP - rows
    b_s = jnp.pad(gt_bboxes[..., 0].reshape(-1), (0, pad_r),
                  constant_values=1.0).reshape(_RP, 1)
    b_e = jnp.pad(gt_bboxes[..., 1].reshape(-1), (0, pad_r),
                  constant_values=0.0).reshape(_RP, 1)
    rowbase = jnp.pad(jnp.repeat(jnp.arange(b, dtype=jnp.int32) * a, g),
                      (0, pad_r)).reshape(_RP, 1)

    grid = a_pad // _CA
    row_spec = pl.BlockSpec((_RP, 1), lambda i: (0, 0))
    outs = pl.pallas_call(
        _argmax_body,
        grid=(grid,),
        in_specs=[
            pl.BlockSpec((1, _CA), lambda i: (0, i)),
            pl.BlockSpec((1, _CA), lambda i: (0, i)),
            row_spec, row_spec, row_spec,
        ],
        out_specs=[row_spec, row_spec, row_spec],
        out_shape=[
            jax.ShapeDtypeStruct((_RP, 1), jnp.float32),
            jax.ShapeDtypeStruct((_RP, 1), jnp.int32),
            jax.ShapeDtypeStruct((_RP, 1), jnp.int32),
        ],
    )(a_s, a_e, b_s, b_e, rowbase)
    return outs[1], outs[2]


def _dense_body(xt_ref, ag_ref, rg_ref, s0_ref, corr_ref, num_anchors):
    x = xt_ref[...].reshape(xt_ref.shape[0] * xt_ref.shape[1],
                            xt_ref.shape[2])
    ag = ag_ref[...]
    rg = rg_ref[...]
    nrows = ag.shape[1]
    chunk = 1024
    y = jnp.zeros((x.shape[0], nrows), jnp.float32)
    ssum = jnp.float32(0.0)
    for s in range(0, num_anchors, chunk):
        n = min(chunk, num_anchors - s)
        xc = x[:, s:s + n]
        ssum += jnp.sum(jnp.maximum(xc, 0.0)
                        + jnp.log1p(jnp.exp(-jnp.abs(xc))))
        ioc = lax.broadcasted_iota(jnp.int32, (n, nrows), 0) + s
        h = jnp.where(ioc == ag, 1.0, 0.0)
        y = y + lax.dot_general(
            xc.astype(jnp.bfloat16), h.astype(jnp.bfloat16),
            (((1,), (0,)), ((), ())),
            preferred_element_type=jnp.float32)
    rio = lax.broadcasted_iota(jnp.int32, (x.shape[0], nrows), 0)
    rmask = jnp.where(rio == rg, 1.0, 0.0)
    s0_ref[0, 0] = ssum
    corr_ref[0, 0] = jnp.sum(y * rmask)


def _dense_loss(scores, gt_classes, a_row):
    b, a, c = scores.shape
    xt = jnp.transpose(scores, (2, 0, 1))
    rows = b * gt_classes.shape[1]
    rg = (gt_classes.astype(jnp.int32) * b
          + jnp.arange(b, dtype=jnp.int32)[:, None]).reshape(1, rows)
    outs = pl.pallas_call(
        functools.partial(_dense_body, num_anchors=a),
        in_specs=[
            pl.BlockSpec((c, b, a), lambda: (0, 0, 0)),
            pl.BlockSpec((1, rows), lambda: (0, 0)),
            pl.BlockSpec((1, rows), lambda: (0, 0)),
        ],
        out_specs=[pl.BlockSpec(memory_space=pltpu.SMEM),
                   pl.BlockSpec(memory_space=pltpu.SMEM)],
        out_shape=[jax.ShapeDtypeStruct((1, 1), jnp.float32),
                   jax.ShapeDtypeStruct((1, 1), jnp.float32)],
    )(xt, a_row, rg)
    return outs[0][0, 0], outs[1][0, 0]


def _sc_body(bflat, bidx0, bidx1, gs, ge, keep, outr,
             b0i_v, b1i_v, gs_v, ge_v, kp_v, b0_v, b1_v, accr_v, sem):
    cid = lax.axis_index("c")
    sid = lax.axis_index("s")
    wid = sid * 2 + cid
    base = wid * _PW
    pltpu.sync_copy(bidx0.at[pl.ds(base, _PW)], b0i_v)
    pltpu.sync_copy(bidx1.at[pl.ds(base, _PW)], b1i_v)
    pltpu.sync_copy(gs.at[pl.ds(base, _PW)], gs_v)
    pltpu.sync_copy(ge.at[pl.ds(base, _PW)], ge_v)
    pltpu.sync_copy(keep.at[pl.ds(base, _PW)], kp_v)
    pltpu.async_copy(bflat.at[b0i_v], b0_v, sem).wait()
    pltpu.async_copy(bflat.at[b1i_v], b1_v, sem).wait()
    accr = jnp.zeros((16,), jnp.float32)
    for j in range(_PW // 16):
        sl = pl.ds(j * 16, 16)
        k = kp_v[sl]
        for bv, gv in ((b0_v[sl], gs_v[sl]), (b1_v[sl], ge_v[sl])):
            pred = jnp.where(gv != 0.0, bv, jnp.zeros((16,), jnp.float32))
            d = pred - gv
            ad = jnp.abs(d)
            sl1 = jnp.where(ad < 1.0, 0.5 * d * d, ad - 0.5)
            accr = accr + k * sl1
    accr_v[...] = accr
    pltpu.sync_copy(accr_v, outr.at[wid])


def _sc_regression(bboxes, bidx0, gt_bboxes):
    rows = gt_bboxes.shape[0] * gt_bboxes.shape[1]
    pad = _SC_ITEMS - rows
    bidx0_p = jnp.pad(bidx0.reshape(-1)[:rows], (0, pad))
    bidx1_p = bidx0_p + 1
    gs = jnp.pad(gt_bboxes[..., 0].reshape(-1), (0, pad))
    ge = jnp.pad(gt_bboxes[..., 1].reshape(-1), (0, pad))
    keep = jnp.pad(jnp.ones((rows,), jnp.float32), (0, pad))
    mesh = plsc.VectorSubcoreMesh(core_axis_name="c", subcore_axis_name="s")
    outr = pl.kernel(
        _sc_body,
        out_type=jax.ShapeDtypeStruct((32, 16), jnp.float32),
        mesh=mesh,
        scratch_types=[
            pltpu.VMEM((_PW,), jnp.int32),
            pltpu.VMEM((_PW,), jnp.int32),
            pltpu.VMEM((_PW,), jnp.float32),
            pltpu.VMEM((_PW,), jnp.float32),
            pltpu.VMEM((_PW,), jnp.float32),
            pltpu.VMEM((_PW,), jnp.float32),
            pltpu.VMEM((_PW,), jnp.float32),
            pltpu.VMEM((16,), jnp.float32),
            pltpu.SemaphoreType.DMA,
        ],
    )(bboxes.reshape(-1), bidx0_p, bidx1_p, gs, ge, keep)
    return jnp.sum(outr)


def kernel(scores, bboxes, gt_classes, gt_bboxes, anchors):
    rows = gt_classes.size
    besti, bidx0 = _best_anchor_indices(anchors, gt_bboxes)
    a_row = besti.reshape(-1)[:rows].reshape(1, rows)
    s0, corr_c = _dense_loss(scores, gt_classes, a_row)
    reg = _sc_regression(bboxes, bidx0, gt_bboxes)
    return s0 - corr_c + reg


# speedup vs baseline: 5.5459x; 2.7397x over previous
import functools

import jax
import jax.numpy as jnp
from jax import lax
from jax.experimental import pallas as pl
from jax.experimental.pallas import tpu as pltpu
from jax.experimental.pallas import tpu_sc as plsc

_EPS = 1e-8
_CA = 512
_RP = 832
_SC_ITEMS = 1024
_PW = _SC_ITEMS // 32
_NEG_BIG = -3.0e38


def _argmax_body(a_s_ref, a_e_ref, b_s_ref, b_e_ref, rowbase_ref,
                 bestv_ref, besti_ref, bidx0_ref):
    i = pl.program_id(0)

    @pl.when(i == 0)
    def _init():
        bestv_ref[...] = jnp.full(bestv_ref.shape, _NEG_BIG, jnp.float32)
        besti_ref[...] = jnp.zeros(besti_ref.shape, jnp.int32)

    a_s = a_s_ref[...]
    a_e = a_e_ref[...]
    b_s = b_s_ref[...]
    b_e = b_e_ref[...]
    inter = jnp.maximum(jnp.minimum(a_e, b_e) - jnp.maximum(a_s, b_s), 0.0)
    denom = (a_e - a_s) + (b_e - b_s + _EPS) - inter
    iou = inter / denom
    mx = jnp.max(iou, axis=1, keepdims=True)
    iota = lax.broadcasted_iota(jnp.int32, iou.shape, 1)
    lid = jnp.min(jnp.where(iou == mx, iota, jnp.int32(2 ** 30)),
                  axis=1, keepdims=True) + i * _CA
    upd = mx > bestv_ref[...]
    besti_ref[...] = jnp.where(upd, lid, besti_ref[...])
    bestv_ref[...] = jnp.where(upd, mx, bestv_ref[...])

    @pl.when(i == pl.num_programs(0) - 1)
    def _emit():
        bidx0_ref[...] = (rowbase_ref[...] + besti_ref[...]) * 2


def _best_anchor_indices(anchors, gt_bboxes):
    b, g = gt_bboxes.shape[0], gt_bboxes.shape[1]
    a = anchors.shape[0]
    a_pad = ((a + _CA - 1) // _CA) * _CA
    rows = b * g
    pad_a = a_pad - a
    a_s = jnp.pad(anchors[:, 0], (0, pad_a), constant_values=2.0).reshape(1, a_pad)
    a_e = jnp.pad(anchors[:, 1], (0, pad_a), constant_values=2.0).reshape(1, a_pad)
    pad_r = _RP - rows
    b_s = jnp.pad(gt_bboxes[..., 0].reshape(-1), (0, pad_r),
                  constant_values=1.0).reshape(_RP, 1)
    b_e = jnp.pad(gt_bboxes[..., 1].reshape(-1), (0, pad_r),
                  constant_values=0.0).reshape(_RP, 1)
    rowbase = jnp.pad(jnp.repeat(jnp.arange(b, dtype=jnp.int32) * a, g),
                      (0, pad_r)).reshape(_RP, 1)

    grid = a_pad // _CA
    row_spec = pl.BlockSpec((_RP, 1), lambda i: (0, 0))
    outs = pl.pallas_call(
        _argmax_body,
        grid=(grid,),
        in_specs=[
            pl.BlockSpec((1, _CA), lambda i: (0, i)),
            pl.BlockSpec((1, _CA), lambda i: (0, i)),
            row_spec, row_spec, row_spec,
        ],
        out_specs=[row_spec, row_spec, row_spec],
        out_shape=[
            jax.ShapeDtypeStruct((_RP, 1), jnp.float32),
            jax.ShapeDtypeStruct((_RP, 1), jnp.int32),
            jax.ShapeDtypeStruct((_RP, 1), jnp.int32),
        ],
    )(a_s, a_e, b_s, b_e, rowbase)
    return outs[1], outs[2]


def _dense_body(xt_ref, bt_ref, ag_ref, rg_ref, bg_ref, gs_ref, ge_ref,
                s0_ref, corr_ref, reg_ref, num_anchors):
    x = xt_ref[...].reshape(xt_ref.shape[0] * xt_ref.shape[1],
                            xt_ref.shape[2])
    bt = bt_ref[...]
    nb = bt.shape[0]
    ag = ag_ref[...]
    rg = rg_ref[...]
    bg = bg_ref[...]
    nrows = ag.shape[1]
    chunk = 1024
    y = jnp.zeros((x.shape[0], nrows), jnp.float32)
    v0 = jnp.zeros((nb, nrows), jnp.float32)
    v1 = jnp.zeros((nb, nrows), jnp.float32)
    ssum = jnp.float32(0.0)
    for s in range(0, num_anchors, chunk):
        n = min(chunk, num_anchors - s)
        xc = x[:, s:s + n]
        ssum += jnp.sum(jnp.maximum(xc, 0.0)
                        + jnp.log1p(jnp.exp(-jnp.abs(xc))))
        ioc = lax.broadcasted_iota(jnp.int32, (n, nrows), 0) + s
        h = jnp.where(ioc == ag, 1.0, 0.0).astype(jnp.bfloat16)
        dims = (((1,), (0,)), ((), ()))
        y = y + lax.dot_general(xc.astype(jnp.bfloat16), h, dims,
                                preferred_element_type=jnp.float32)
        v0 = v0 + lax.dot_general(bt[:, 0, s:s + n].astype(jnp.bfloat16), h,
                                  dims, preferred_element_type=jnp.float32)
        v1 = v1 + lax.dot_general(bt[:, 1, s:s + n].astype(jnp.bfloat16), h,
                                  dims, preferred_element_type=jnp.float32)
    rio = lax.broadcasted_iota(jnp.int32, (x.shape[0], nrows), 0)
    rmask = jnp.where(rio == rg, 1.0, 0.0)
    s0_ref[0, 0] = ssum
    corr_ref[0, 0] = jnp.sum(y * rmask)
    bio = lax.broadcasted_iota(jnp.int32, (nb, nrows), 0)
    bmask = jnp.where(bio == bg, 1.0, 0.0)
    p0 = jnp.sum(v0 * bmask, axis=0, keepdims=True)
    p1 = jnp.sum(v1 * bmask, axis=0, keepdims=True)
    reg = jnp.float32(0.0)
    for pv, gv in ((p0, gs_ref[...]), (p1, ge_ref[...])):
        pred = jnp.where(gv != 0.0, pv, 0.0)
        d = pred - gv
        ad = jnp.abs(d)
        reg += jnp.sum(jnp.where(ad < 1.0, 0.5 * d * d, ad - 0.5))
    reg_ref[0, 0] = reg


def _dense_loss(scores, bboxes, gt_classes, gt_bboxes, a_row):
    b, a, c = scores.shape
    xt = jnp.transpose(scores, (2, 0, 1))
    bt = jnp.transpose(bboxes, (0, 2, 1))
    rows = b * gt_classes.shape[1]
    rg = (gt_classes.astype(jnp.int32) * b
          + jnp.arange(b, dtype=jnp.int32)[:, None]).reshape(1, rows)
    bg = jnp.broadcast_to(jnp.arange(b, dtype=jnp.int32)[:, None],
                          gt_classes.shape).reshape(1, rows)
    gs = gt_bboxes[..., 0].reshape(1, rows)
    ge = gt_bboxes[..., 1].reshape(1, rows)
    row_spec = pl.BlockSpec((1, rows), lambda: (0, 0))
    outs = pl.pallas_call(
        functools.partial(_dense_body, num_anchors=a),
        in_specs=[
            pl.BlockSpec((c, b, a), lambda: (0, 0, 0)),
            pl.BlockSpec((b, 2, a), lambda: (0, 0, 0)),
            row_spec, row_spec, row_spec, row_spec, row_spec,
        ],
        out_specs=[pl.BlockSpec(memory_space=pltpu.SMEM),
                   pl.BlockSpec(memory_space=pltpu.SMEM),
                   pl.BlockSpec(memory_space=pltpu.SMEM)],
        out_shape=[jax.ShapeDtypeStruct((1, 1), jnp.float32),
                   jax.ShapeDtypeStruct((1, 1), jnp.float32),
                   jax.ShapeDtypeStruct((1, 1), jnp.float32)],
    )(xt, bt, a_row, rg, bg, gs, ge)
    return outs[0][0, 0], outs[1][0, 0], outs[2][0, 0]


def _sc_body(bflat, bidx0, bidx1, gs, ge, keep, outr,
             b0i_v, b1i_v, gs_v, ge_v, kp_v, b0_v, b1_v, accr_v, sem):
    cid = lax.axis_index("c")
    sid = lax.axis_index("s")
    wid = sid * 2 + cid
    base = wid * _PW
    pltpu.sync_copy(bidx0.at[pl.ds(base, _PW)], b0i_v)
    pltpu.sync_copy(bidx1.at[pl.ds(base, _PW)], b1i_v)
    pltpu.sync_copy(gs.at[pl.ds(base, _PW)], gs_v)
    pltpu.sync_copy(ge.at[pl.ds(base, _PW)], ge_v)
    pltpu.sync_copy(keep.at[pl.ds(base, _PW)], kp_v)
    pltpu.async_copy(bflat.at[b0i_v], b0_v, sem).wait()
    pltpu.async_copy(bflat.at[b1i_v], b1_v, sem).wait()
    accr = jnp.zeros((16,), jnp.float32)
    for j in range(_PW // 16):
        sl = pl.ds(j * 16, 16)
        k = kp_v[sl]
        for bv, gv in ((b0_v[sl], gs_v[sl]), (b1_v[sl], ge_v[sl])):
            pred = jnp.where(gv != 0.0, bv, jnp.zeros((16,), jnp.float32))
            d = pred - gv
            ad = jnp.abs(d)
            sl1 = jnp.where(ad < 1.0, 0.5 * d * d, ad - 0.5)
            accr = accr + k * sl1
    accr_v[...] = accr
    pltpu.sync_copy(accr_v, outr.at[wid])


def _sc_regression(bboxes, bidx0, gt_bboxes):
    rows = gt_bboxes.shape[0] * gt_bboxes.shape[1]
    pad = _SC_ITEMS - rows
    bidx0_p = jnp.pad(bidx0.reshape(-1)[:rows], (0, pad))
    bidx1_p = bidx0_p + 1
    gs = jnp.pad(gt_bboxes[..., 0].reshape(-1), (0, pad))
    ge = jnp.pad(gt_bboxes[..., 1].reshape(-1), (0, pad))
    keep = jnp.pad(jnp.ones((rows,), jnp.float32), (0, pad))
    mesh = plsc.VectorSubcoreMesh(core_axis_name="c", subcore_axis_name="s")
    outr = pl.kernel(
        _sc_body,
        out_type=jax.ShapeDtypeStruct((32, 16), jnp.float32),
        mesh=mesh,
        scratch_types=[
            pltpu.VMEM((_PW,), jnp.int32),
            pltpu.VMEM((_PW,), jnp.int32),
            pltpu.VMEM((_PW,), jnp.float32),
            pltpu.VMEM((_PW,), jnp.float32),
            pltpu.VMEM((_PW,), jnp.float32),
            pltpu.VMEM((_PW,), jnp.float32),
            pltpu.VMEM((_PW,), jnp.float32),
            pltpu.VMEM((16,), jnp.float32),
            pltpu.SemaphoreType.DMA,
        ],
    )(bboxes.reshape(-1), bidx0_p, bidx1_p, gs, ge, keep)
    return jnp.sum(outr)


def kernel(scores, bboxes, gt_classes, gt_bboxes, anchors):
    rows = gt_classes.size
    besti, bidx0 = _best_anchor_indices(anchors, gt_bboxes)
    a_row = besti.reshape(-1)[:rows].reshape(1, rows)
    s0, corr_c, reg = _dense_loss(scores, bboxes, gt_classes, gt_bboxes,
                                  a_row)
    return s0 - corr_c + reg


# speedup vs baseline: 5.6714x; 1.0226x over previous
import functools

import jax
import jax.numpy as jnp
from jax import lax
from jax.experimental import pallas as pl
from jax.experimental.pallas import tpu as pltpu
from jax.experimental.pallas import tpu_sc as plsc

_EPS = 1e-8
_CA = 512
_RP = 832
_SC_ITEMS = 1024
_PW = _SC_ITEMS // 32
_NEG_BIG = -3.0e38


def _argmax_body(a_s_ref, a_e_ref, b_s_ref, b_e_ref, rowbase_ref,
                 bestk_ref, besti_ref, bidx0_ref):
    i = pl.program_id(0)

    @pl.when(i == 0)
    def _init():
        bestk_ref[...] = jnp.zeros(bestk_ref.shape, jnp.int32)

    a_s = a_s_ref[...]
    a_e = a_e_ref[...]
    b_s = b_s_ref[...]
    b_e = b_e_ref[...]
    inter = jnp.maximum(jnp.minimum(a_e, b_e) - jnp.maximum(a_s, b_s), 0.0)
    denom = (a_e - a_s) + (b_e - b_s + _EPS) - inter
    iou = inter / denom
    bits = lax.bitcast_convert_type(iou, jnp.int32)
    lowb = (jnp.int32(0x7FFF)
            - lax.broadcasted_iota(jnp.int32, (1, _CA), 1)
            - jnp.int32(i * _CA))
    key = (bits & jnp.int32(0x7FFF8000)) | lowb
    mx = jnp.max(key, axis=1, keepdims=True)
    bestk_ref[...] = jnp.maximum(bestk_ref[...], mx)

    @pl.when(i == pl.num_programs(0) - 1)
    def _emit():
        besti = jnp.int32(0x7FFF) - (bestk_ref[...] & jnp.int32(0x7FFF))
        besti_ref[...] = besti
        bidx0_ref[...] = (rowbase_ref[...] + besti) * 2


def _best_anchor_indices(anchors, gt_bboxes):
    b, g = gt_bboxes.shape[0], gt_bboxes.shape[1]
    a = anchors.shape[0]
    a_pad = ((a + _CA - 1) // _CA) * _CA
    rows = b * g
    pad_a = a_pad - a
    a_s = jnp.pad(anchors[:, 0], (0, pad_a), constant_values=2.0).reshape(1, a_pad)
    a_e = jnp.pad(anchors[:, 1], (0, pad_a), constant_values=2.0).reshape(1, a_pad)
    pad_r = _RP - rows
    b_s = jnp.pad(gt_bboxes[..., 0].reshape(-1), (0, pad_r),
                  constant_values=1.0).reshape(_RP, 1)
    b_e = jnp.pad(gt_bboxes[..., 1].reshape(-1), (0, pad_r),
                  constant_values=0.0).reshape(_RP, 1)
    rowbase = jnp.pad(jnp.repeat(jnp.arange(b, dtype=jnp.int32) * a, g),
                      (0, pad_r)).reshape(_RP, 1)

    grid = a_pad // _CA
    row_spec = pl.BlockSpec((_RP, 1), lambda i: (0, 0))
    outs = pl.pallas_call(
        _argmax_body,
        grid=(grid,),
        in_specs=[
            pl.BlockSpec((1, _CA), lambda i: (0, i)),
            pl.BlockSpec((1, _CA), lambda i: (0, i)),
            row_spec, row_spec, row_spec,
        ],
        out_specs=[row_spec, row_spec, row_spec],
        out_shape=[
            jax.ShapeDtypeStruct((_RP, 1), jnp.int32),
            jax.ShapeDtypeStruct((_RP, 1), jnp.int32),
            jax.ShapeDtypeStruct((_RP, 1), jnp.int32),
        ],
    )(a_s, a_e, b_s, b_e, rowbase)
    return outs[1], outs[2]


def _dense_body(xt_ref, bt_ref, ag_ref, rg_ref, bg_ref, gs_ref, ge_ref,
                s0_ref, corr_ref, reg_ref, num_anchors):
    x = xt_ref[...].reshape(xt_ref.shape[0] * xt_ref.shape[1],
                            xt_ref.shape[2])
    bt = bt_ref[...]
    nb = bt.shape[0]
    ag = ag_ref[...]
    rg = rg_ref[...]
    bg = bg_ref[...]
    nrows = ag.shape[1]
    chunk = 1024
    y = jnp.zeros((x.shape[0], nrows), jnp.float32)
    v0 = jnp.zeros((nb, nrows), jnp.float32)
    v1 = jnp.zeros((nb, nrows), jnp.float32)
    ssum = jnp.float32(0.0)
    for s in range(0, num_anchors, chunk):
        n = min(chunk, num_anchors - s)
        xc = x[:, s:s + n]
        ssum += jnp.sum(jnp.maximum(xc, 0.0)
                        + jnp.log1p(jnp.exp(-jnp.abs(xc))))
        ioc = lax.broadcasted_iota(jnp.int32, (n, nrows), 0) + s
        h = jnp.where(ioc == ag, 1.0, 0.0).astype(jnp.bfloat16)
        dims = (((1,), (0,)), ((), ()))
        y = y + lax.dot_general(xc.astype(jnp.bfloat16), h, dims,
                                preferred_element_type=jnp.float32)
        v0 = v0 + lax.dot_general(bt[:, 0, s:s + n].astype(jnp.bfloat16), h,
                                  dims, preferred_element_type=jnp.float32)
        v1 = v1 + lax.dot_general(bt[:, 1, s:s + n].astype(jnp.bfloat16), h,
                                  dims, preferred_element_type=jnp.float32)
    rio = lax.broadcasted_iota(jnp.int32, (x.shape[0], nrows), 0)
    rmask = jnp.where(rio == rg, 1.0, 0.0)
    s0_ref[0, 0] = ssum
    corr_ref[0, 0] = jnp.sum(y * rmask)
    bio = lax.broadcasted_iota(jnp.int32, (nb, nrows), 0)
    bmask = jnp.where(bio == bg, 1.0, 0.0)
    p0 = jnp.sum(v0 * bmask, axis=0, keepdims=True)
    p1 = jnp.sum(v1 * bmask, axis=0, keepdims=True)
    reg = jnp.float32(0.0)
    for pv, gv in ((p0, gs_ref[...]), (p1, ge_ref[...])):
        pred = jnp.where(gv != 0.0, pv, 0.0)
        d = pred - gv
        ad = jnp.abs(d)
        reg += jnp.sum(jnp.where(ad < 1.0, 0.5 * d * d, ad - 0.5))
    reg_ref[0, 0] = reg


def _dense_loss(scores, bboxes, gt_classes, gt_bboxes, a_row):
    b, a, c = scores.shape
    xt = jnp.transpose(scores, (2, 0, 1))
    bt = jnp.transpose(bboxes, (0, 2, 1))
    rows = b * gt_classes.shape[1]
    rg = (gt_classes.astype(jnp.int32) * b
          + jnp.arange(b, dtype=jnp.int32)[:, None]).reshape(1, rows)
    bg = jnp.broadcast_to(jnp.arange(b, dtype=jnp.int32)[:, None],
                          gt_classes.shape).reshape(1, rows)
    gs = gt_bboxes[..., 0].reshape(1, rows)
    ge = gt_bboxes[..., 1].reshape(1, rows)
    row_spec = pl.BlockSpec((1, rows), lambda: (0, 0))
    outs = pl.pallas_call(
        functools.partial(_dense_body, num_anchors=a),
        in_specs=[
            pl.BlockSpec((c, b, a), lambda: (0, 0, 0)),
            pl.BlockSpec((b, 2, a), lambda: (0, 0, 0)),
            row_spec, row_spec, row_spec, row_spec, row_spec,
        ],
        out_specs=[pl.BlockSpec(memory_space=pltpu.SMEM),
                   pl.BlockSpec(memory_space=pltpu.SMEM),
                   pl.BlockSpec(memory_space=pltpu.SMEM)],
        out_shape=[jax.ShapeDtypeStruct((1, 1), jnp.float32),
                   jax.ShapeDtypeStruct((1, 1), jnp.float32),
                   jax.ShapeDtypeStruct((1, 1), jnp.float32)],
    )(xt, bt, a_row, rg, bg, gs, ge)
    return outs[0][0, 0], outs[1][0, 0], outs[2][0, 0]


def _sc_body(bflat, bidx0, bidx1, gs, ge, keep, outr,
             b0i_v, b1i_v, gs_v, ge_v, kp_v, b0_v, b1_v, accr_v, sem):
    cid = lax.axis_index("c")
    sid = lax.axis_index("s")
    wid = sid * 2 + cid
    base = wid * _PW
    pltpu.sync_copy(bidx0.at[pl.ds(base, _PW)], b0i_v)
    pltpu.sync_copy(bidx1.at[pl.ds(base, _PW)], b1i_v)
    pltpu.sync_copy(gs.at[pl.ds(base, _PW)], gs_v)
    pltpu.sync_copy(ge.at[pl.ds(base, _PW)], ge_v)
    pltpu.sync_copy(keep.at[pl.ds(base, _PW)], kp_v)
    pltpu.async_copy(bflat.at[b0i_v], b0_v, sem).wait()
    pltpu.async_copy(bflat.at[b1i_v], b1_v, sem).wait()
    accr = jnp.zeros((16,), jnp.float32)
    for j in range(_PW // 16):
        sl = pl.ds(j * 16, 16)
        k = kp_v[sl]
        for bv, gv in ((b0_v[sl], gs_v[sl]), (b1_v[sl], ge_v[sl])):
            pred = jnp.where(gv != 0.0, bv, jnp.zeros((16,), jnp.float32))
            d = pred - gv
            ad = jnp.abs(d)
            sl1 = jnp.where(ad < 1.0, 0.5 * d * d, ad - 0.5)
            accr = accr + k * sl1
    accr_v[...] = accr
    pltpu.sync_copy(accr_v, outr.at[wid])


def _sc_regression(bboxes, bidx0, gt_bboxes):
    rows = gt_bboxes.shape[0] * gt_bboxes.shape[1]
    pad = _SC_ITEMS - rows
    bidx0_p = jnp.pad(bidx0.reshape(-1)[:rows], (0, pad))
    bidx1_p = bidx0_p + 1
    gs = jnp.pad(gt_bboxes[..., 0].reshape(-1), (0, pad))
    ge = jnp.pad(gt_bboxes[..., 1].reshape(-1), (0, pad))
    keep = jnp.pad(jnp.ones((rows,), jnp.float32), (0, pad))
    mesh = plsc.VectorSubcoreMesh(core_axis_name="c", subcore_axis_name="s")
    outr = pl.kernel(
        _sc_body,
        out_type=jax.ShapeDtypeStruct((32, 16), jnp.float32),
        mesh=mesh,
        scratch_types=[
            pltpu.VMEM((_PW,), jnp.int32),
            pltpu.VMEM((_PW,), jnp.int32),
            pltpu.VMEM((_PW,), jnp.float32),
            pltpu.VMEM((_PW,), jnp.float32),
            pltpu.VMEM((_PW,), jnp.float32),
            pltpu.VMEM((_PW,), jnp.float32),
            pltpu.VMEM((_PW,), jnp.float32),
            pltpu.VMEM((16,), jnp.float32),
            pltpu.SemaphoreType.DMA,
        ],
    )(bboxes.reshape(-1), bidx0_p, bidx1_p, gs, ge, keep)
    return jnp.sum(outr)


def kernel(scores, bboxes, gt_classes, gt_bboxes, anchors):
    rows = gt_classes.size
    besti, bidx0 = _best_anchor_indices(anchors, gt_bboxes)
    a_row = besti.reshape(-1)[:rows].reshape(1, rows)
    s0, corr_c, reg = _dense_loss(scores, bboxes, gt_classes, gt_bboxes,
                                  a_row)
    return s0 - corr_c + reg


# speedup vs baseline: 6.6044x; 1.1645x over previous
import functools

import jax
import jax.numpy as jnp
from jax import lax
from jax.experimental import pallas as pl
from jax.experimental.pallas import tpu as pltpu
from jax.experimental.pallas import tpu_sc as plsc

_EPS = 1e-8
_CA = 2048
_RP = 832
_SC_ITEMS = 1024
_PW = _SC_ITEMS // 32
_NEG_BIG = -3.0e38


def _argmax_body(a_s_ref, a_e_ref, b_s_ref, b_e_ref, rowbase_ref,
                 bestk_ref, besti_ref, bidx0_ref):
    i = pl.program_id(0)

    @pl.when(i == 0)
    def _init():
        bestk_ref[...] = jnp.zeros(bestk_ref.shape, jnp.int32)

    a_s = a_s_ref[...]
    a_e = a_e_ref[...]
    b_s = b_s_ref[...]
    b_e = b_e_ref[...]
    inter = jnp.maximum(jnp.minimum(a_e, b_e) - jnp.maximum(a_s, b_s), 0.0)
    denom = (a_e - a_s) + (b_e - b_s + _EPS) - inter
    iou = inter / denom
    bits = lax.bitcast_convert_type(iou, jnp.int32)
    lowb = (jnp.int32(0x7FFF)
            - lax.broadcasted_iota(jnp.int32, (1, _CA), 1)
            - jnp.int32(i * _CA))
    key = (bits & jnp.int32(0x7FFF8000)) | lowb
    mx = jnp.max(key, axis=1, keepdims=True)
    bestk_ref[...] = jnp.maximum(bestk_ref[...], mx)

    @pl.when(i == pl.num_programs(0) - 1)
    def _emit():
        besti = jnp.int32(0x7FFF) - (bestk_ref[...] & jnp.int32(0x7FFF))
        besti_ref[...] = besti
        bidx0_ref[...] = (rowbase_ref[...] + besti) * 2


def _best_anchor_indices(anchors, gt_bboxes):
    b, g = gt_bboxes.shape[0], gt_bboxes.shape[1]
    a = anchors.shape[0]
    a_pad = ((a + _CA - 1) // _CA) * _CA
    rows = b * g
    pad_a = a_pad - a
    a_s = jnp.pad(anchors[:, 0], (0, pad_a), constant_values=2.0).reshape(1, a_pad)
    a_e = jnp.pad(anchors[:, 1], (0, pad_a), constant_values=2.0).reshape(1, a_pad)
    pad_r = _RP - rows
    b_s = jnp.pad(gt_bboxes[..., 0].reshape(-1), (0, pad_r),
                  constant_values=1.0).reshape(_RP, 1)
    b_e = jnp.pad(gt_bboxes[..., 1].reshape(-1), (0, pad_r),
                  constant_values=0.0).reshape(_RP, 1)
    rowbase = jnp.pad(jnp.repeat(jnp.arange(b, dtype=jnp.int32) * a, g),
                      (0, pad_r)).reshape(_RP, 1)

    grid = a_pad // _CA
    row_spec = pl.BlockSpec((_RP, 1), lambda i: (0, 0))
    outs = pl.pallas_call(
        _argmax_body,
        grid=(grid,),
        in_specs=[
            pl.BlockSpec((1, _CA), lambda i: (0, i)),
            pl.BlockSpec((1, _CA), lambda i: (0, i)),
            row_spec, row_spec, row_spec,
        ],
        out_specs=[row_spec, row_spec, row_spec],
        out_shape=[
            jax.ShapeDtypeStruct((_RP, 1), jnp.int32),
            jax.ShapeDtypeStruct((_RP, 1), jnp.int32),
            jax.ShapeDtypeStruct((_RP, 1), jnp.int32),
        ],
    )(a_s, a_e, b_s, b_e, rowbase)
    return outs[1], outs[2]


def _dense_body(xt_ref, bt_ref, ag_ref, rg_ref, bg_ref, gs_ref, ge_ref,
                s0_ref, corr_ref, reg_ref, num_anchors):
    x = xt_ref[...].reshape(xt_ref.shape[0] * xt_ref.shape[1],
                            xt_ref.shape[2])
    bt = bt_ref[...]
    nb = bt.shape[0]
    ag = ag_ref[...]
    rg = rg_ref[...]
    bg = bg_ref[...]
    nrows = ag.shape[1]
    chunk = 1024
    y = jnp.zeros((x.shape[0], nrows), jnp.float32)
    v0 = jnp.zeros((nb, nrows), jnp.float32)
    v1 = jnp.zeros((nb, nrows), jnp.float32)
    ssum = jnp.float32(0.0)
    for s in range(0, num_anchors, chunk):
        n = min(chunk, num_anchors - s)
        xc = x[:, s:s + n]
        ssum += jnp.sum(jnp.maximum(xc, 0.0)
                        + jnp.log1p(jnp.exp(-jnp.abs(xc))))
        ioc = lax.broadcasted_iota(jnp.int32, (n, nrows), 0) + s
        h = jnp.where(ioc == ag, 1.0, 0.0).astype(jnp.bfloat16)
        dims = (((1,), (0,)), ((), ()))
        y = y + lax.dot_general(xc.astype(jnp.bfloat16), h, dims,
                                preferred_element_type=jnp.float32)
        v0 = v0 + lax.dot_general(bt[:, 0, s:s + n].astype(jnp.bfloat16), h,
                                  dims, preferred_element_type=jnp.float32)
        v1 = v1 + lax.dot_general(bt[:, 1, s:s + n].astype(jnp.bfloat16), h,
                                  dims, preferred_element_type=jnp.float32)
    rio = lax.broadcasted_iota(jnp.int32, (x.shape[0], nrows), 0)
    rmask = jnp.where(rio == rg, 1.0, 0.0)
    s0_ref[0, 0] = ssum
    corr_ref[0, 0] = jnp.sum(y * rmask)
    bio = lax.broadcasted_iota(jnp.int32, (nb, nrows), 0)
    bmask = jnp.where(bio == bg, 1.0, 0.0)
    p0 = jnp.sum(v0 * bmask, axis=0, keepdims=True)
    p1 = jnp.sum(v1 * bmask, axis=0, keepdims=True)
    reg = jnp.float32(0.0)
    for pv, gv in ((p0, gs_ref[...]), (p1, ge_ref[...])):
        pred = jnp.where(gv != 0.0, pv, 0.0)
        d = pred - gv
        ad = jnp.abs(d)
        reg += jnp.sum(jnp.where(ad < 1.0, 0.5 * d * d, ad - 0.5))
    reg_ref[0, 0] = reg


def _dense_loss(scores, bboxes, gt_classes, gt_bboxes, a_row):
    b, a, c = scores.shape
    xt = jnp.transpose(scores, (2, 0, 1))
    bt = jnp.transpose(bboxes, (0, 2, 1))
    rows = b * gt_classes.shape[1]
    rg = (gt_classes.astype(jnp.int32) * b
          + jnp.arange(b, dtype=jnp.int32)[:, None]).reshape(1, rows)
    bg = jnp.broadcast_to(jnp.arange(b, dtype=jnp.int32)[:, None],
                          gt_classes.shape).reshape(1, rows)
    gs = gt_bboxes[..., 0].reshape(1, rows)
    ge = gt_bboxes[..., 1].reshape(1, rows)
    row_spec = pl.BlockSpec((1, rows), lambda: (0, 0))
    outs = pl.pallas_call(
        functools.partial(_dense_body, num_anchors=a),
        in_specs=[
            pl.BlockSpec((c, b, a), lambda: (0, 0, 0)),
            pl.BlockSpec((b, 2, a), lambda: (0, 0, 0)),
            row_spec, row_spec, row_spec, row_spec, row_spec,
        ],
        out_specs=[pl.BlockSpec(memory_space=pltpu.SMEM),
                   pl.BlockSpec(memory_space=pltpu.SMEM),
                   pl.BlockSpec(memory_space=pltpu.SMEM)],
        out_shape=[jax.ShapeDtypeStruct((1, 1), jnp.float32),
                   jax.ShapeDtypeStruct((1, 1), jnp.float32),
                   jax.ShapeDtypeStruct((1, 1), jnp.float32)],
    )(xt, bt, a_row, rg, bg, gs, ge)
    return outs[0][0, 0], outs[1][0, 0], outs[2][0, 0]


def _sc_body(bflat, bidx0, bidx1, gs, ge, keep, outr,
             b0i_v, b1i_v, gs_v, ge_v, kp_v, b0_v, b1_v, accr_v, sem):
    cid = lax.axis_index("c")
    sid = lax.axis_index("s")
    wid = sid * 2 + cid
    base = wid * _PW
    pltpu.sync_copy(bidx0.at[pl.ds(base, _PW)], b0i_v)
    pltpu.sync_copy(bidx1.at[pl.ds(base, _PW)], b1i_v)
    pltpu.sync_copy(gs.at[pl.ds(base, _PW)], gs_v)
    pltpu.sync_copy(ge.at[pl.ds(base, _PW)], ge_v)
    pltpu.sync_copy(keep.at[pl.ds(base, _PW)], kp_v)
    pltpu.async_copy(bflat.at[b0i_v], b0_v, sem).wait()
    pltpu.async_copy(bflat.at[b1i_v], b1_v, sem).wait()
    accr = jnp.zeros((16,), jnp.float32)
    for j in range(_PW // 16):
        sl = pl.ds(j * 16, 16)
        k = kp_v[sl]
        for bv, gv in ((b0_v[sl], gs_v[sl]), (b1_v[sl], ge_v[sl])):
            pred = jnp.where(gv != 0.0, bv, jnp.zeros((16,), jnp.float32))
            d = pred - gv
            ad = jnp.abs(d)
            sl1 = jnp.where(ad < 1.0, 0.5 * d * d, ad - 0.5)
            accr = accr + k * sl1
    accr_v[...] = accr
    pltpu.sync_copy(accr_v, outr.at[wid])


def _sc_regression(bboxes, bidx0, gt_bboxes):
    rows = gt_bboxes.shape[0] * gt_bboxes.shape[1]
    pad = _SC_ITEMS - rows
    bidx0_p = jnp.pad(bidx0.reshape(-1)[:rows], (0, pad))
    bidx1_p = bidx0_p + 1
    gs = jnp.pad(gt_bboxes[..., 0].reshape(-1), (0, pad))
    ge = jnp.pad(gt_bboxes[..., 1].reshape(-1), (0, pad))
    keep = jnp.pad(jnp.ones((rows,), jnp.float32), (0, pad))
    mesh = plsc.VectorSubcoreMesh(core_axis_name="c", subcore_axis_name="s")
    outr = pl.kernel(
        _sc_body,
        out_type=jax.ShapeDtypeStruct((32, 16), jnp.float32),
        mesh=mesh,
        scratch_types=[
            pltpu.VMEM((_PW,), jnp.int32),
            pltpu.VMEM((_PW,), jnp.int32),
            pltpu.VMEM((_PW,), jnp.float32),
            pltpu.VMEM((_PW,), jnp.float32),
            pltpu.VMEM((_PW,), jnp.float32),
            pltpu.VMEM((_PW,), jnp.float32),
            pltpu.VMEM((_PW,), jnp.float32),
            pltpu.VMEM((16,), jnp.float32),
            pltpu.SemaphoreType.DMA,
        ],
    )(bboxes.reshape(-1), bidx0_p, bidx1_p, gs, ge, keep)
    return jnp.sum(outr)


def kernel(scores, bboxes, gt_classes, gt_bboxes, anchors):
    rows = gt_classes.size
    besti, bidx0 = _best_anchor_indices(anchors, gt_bboxes)
    a_row = besti.reshape(-1)[:rows].reshape(1, rows)
    s0, corr_c, reg = _dense_loss(scores, bboxes, gt_classes, gt_bboxes,
                                  a_row)
    return s0 - corr_c + reg


# speedup vs baseline: 6.7162x; 1.0169x over previous
import functools

import jax
import jax.numpy as jnp
from jax import lax
from jax.experimental import pallas as pl
from jax.experimental.pallas import tpu as pltpu
from jax.experimental.pallas import tpu_sc as plsc

_EPS = 1e-8
_CA = 4096
_RP = 800
_SC_ITEMS = 1024
_PW = _SC_ITEMS // 32
_NEG_BIG = -3.0e38


def _argmax_body(a_s_ref, a_e_ref, b_s_ref, b_e_ref, rowbase_ref,
                 bestk_ref, besti_ref, bidx0_ref):
    i = pl.program_id(0)

    @pl.when(i == 0)
    def _init():
        bestk_ref[...] = jnp.zeros(bestk_ref.shape, jnp.int32)

    a_s = a_s_ref[...]
    a_e = a_e_ref[...]
    b_s = b_s_ref[...]
    b_e = b_e_ref[...]
    inter = jnp.maximum(jnp.minimum(a_e, b_e) - jnp.maximum(a_s, b_s), 0.0)
    denom = (a_e - a_s) + (b_e - b_s + _EPS) - inter
    iou = inter / denom
    bits = lax.bitcast_convert_type(iou, jnp.int32)
    lowb = (jnp.int32(0x7FFF)
            - lax.broadcasted_iota(jnp.int32, (1, _CA), 1)
            - jnp.int32(i * _CA))
    key = (bits & jnp.int32(0x7FFF8000)) | lowb
    mx = jnp.max(key, axis=1, keepdims=True)
    bestk_ref[...] = jnp.maximum(bestk_ref[...], mx)

    @pl.when(i == pl.num_programs(0) - 1)
    def _emit():
        besti = jnp.int32(0x7FFF) - (bestk_ref[...] & jnp.int32(0x7FFF))
        besti_ref[...] = besti
        bidx0_ref[...] = (rowbase_ref[...] + besti) * 2


def _best_anchor_indices(anchors, gt_bboxes):
    b, g = gt_bboxes.shape[0], gt_bboxes.shape[1]
    a = anchors.shape[0]
    a_pad = ((a + _CA - 1) // _CA) * _CA
    rows = b * g
    pad_a = a_pad - a
    a_s = jnp.pad(anchors[:, 0], (0, pad_a), constant_values=2.0).reshape(1, a_pad)
    a_e = jnp.pad(anchors[:, 1], (0, pad_a), constant_values=2.0).reshape(1, a_pad)
    pad_r = _RP - rows
    b_s = jnp.pad(gt_bboxes[..., 0].reshape(-1), (0, pad_r),
                  constant_values=1.0).reshape(_RP, 1)
    b_e = jnp.pad(gt_bboxes[..., 1].reshape(-1), (0, pad_r),
                  constant_values=0.0).reshape(_RP, 1)
    rowbase = jnp.pad(jnp.repeat(jnp.arange(b, dtype=jnp.int32) * a, g),
                      (0, pad_r)).reshape(_RP, 1)

    grid = a_pad // _CA
    row_spec = pl.BlockSpec((_RP, 1), lambda i: (0, 0))
    outs = pl.pallas_call(
        _argmax_body,
        grid=(grid,),
        in_specs=[
            pl.BlockSpec((1, _CA), lambda i: (0, i)),
            pl.BlockSpec((1, _CA), lambda i: (0, i)),
            row_spec, row_spec, row_spec,
        ],
        out_specs=[row_spec, row_spec, row_spec],
        out_shape=[
            jax.ShapeDtypeStruct((_RP, 1), jnp.int32),
            jax.ShapeDtypeStruct((_RP, 1), jnp.int32),
            jax.ShapeDtypeStruct((_RP, 1), jnp.int32),
        ],
    )(a_s, a_e, b_s, b_e, rowbase)
    return outs[1], outs[2]


def _dense_body(xt_ref, bt_ref, ag_ref, rg_ref, bg_ref, gs_ref, ge_ref,
                s0_ref, corr_ref, reg_ref, num_anchors):
    x = xt_ref[...].reshape(xt_ref.shape[0] * xt_ref.shape[1],
                            xt_ref.shape[2])
    bt = bt_ref[...]
    nb = bt.shape[0]
    ag = ag_ref[...]
    rg = rg_ref[...]
    bg = bg_ref[...]
    nrows = ag.shape[1]
    chunk = 1024
    y = jnp.zeros((x.shape[0], nrows), jnp.float32)
    v0 = jnp.zeros((nb, nrows), jnp.float32)
    v1 = jnp.zeros((nb, nrows), jnp.float32)
    ssum = jnp.float32(0.0)
    for s in range(0, num_anchors, chunk):
        n = min(chunk, num_anchors - s)
        xc = x[:, s:s + n]
        ssum += jnp.sum(jnp.maximum(xc, 0.0)
                        + jnp.log1p(jnp.exp(-jnp.abs(xc))))
        ioc = lax.broadcasted_iota(jnp.int32, (n, nrows), 0) + s
        h = jnp.where(ioc == ag, 1.0, 0.0).astype(jnp.bfloat16)
        dims = (((1,), (0,)), ((), ()))
        y = y + lax.dot_general(xc.astype(jnp.bfloat16), h, dims,
                                preferred_element_type=jnp.float32)
        v0 = v0 + lax.dot_general(bt[:, 0, s:s + n].astype(jnp.bfloat16), h,
                                  dims, preferred_element_type=jnp.float32)
        v1 = v1 + lax.dot_general(bt[:, 1, s:s + n].astype(jnp.bfloat16), h,
                                  dims, preferred_element_type=jnp.float32)
    rio = lax.broadcasted_iota(jnp.int32, (x.shape[0], nrows), 0)
    rmask = jnp.where(rio == rg, 1.0, 0.0)
    s0_ref[0, 0] = ssum
    corr_ref[0, 0] = jnp.sum(y * rmask)
    bio = lax.broadcasted_iota(jnp.int32, (nb, nrows), 0)
    bmask = jnp.where(bio == bg, 1.0, 0.0)
    p0 = jnp.sum(v0 * bmask, axis=0, keepdims=True)
    p1 = jnp.sum(v1 * bmask, axis=0, keepdims=True)
    reg = jnp.float32(0.0)
    for pv, gv in ((p0, gs_ref[...]), (p1, ge_ref[...])):
        pred = jnp.where(gv != 0.0, pv, 0.0)
        d = pred - gv
        ad = jnp.abs(d)
        reg += jnp.sum(jnp.where(ad < 1.0, 0.5 * d * d, ad - 0.5))
    reg_ref[0, 0] = reg


def _dense_loss(scores, bboxes, gt_classes, gt_bboxes, a_row):
    b, a, c = scores.shape
    xt = jnp.transpose(scores, (2, 0, 1))
    bt = jnp.transpose(bboxes, (0, 2, 1))
    rows = b * gt_classes.shape[1]
    rg = (gt_classes.astype(jnp.int32) * b
          + jnp.arange(b, dtype=jnp.int32)[:, None]).reshape(1, rows)
    bg = jnp.broadcast_to(jnp.arange(b, dtype=jnp.int32)[:, None],
                          gt_classes.shape).reshape(1, rows)
    gs = gt_bboxes[..., 0].reshape(1, rows)
    ge = gt_bboxes[..., 1].reshape(1, rows)
    row_spec = pl.BlockSpec((1, rows), lambda: (0, 0))
    outs = pl.pallas_call(
        functools.partial(_dense_body, num_anchors=a),
        in_specs=[
            pl.BlockSpec((c, b, a), lambda: (0, 0, 0)),
            pl.BlockSpec((b, 2, a), lambda: (0, 0, 0)),
            row_spec, row_spec, row_spec, row_spec, row_spec,
        ],
        out_specs=[pl.BlockSpec(memory_space=pltpu.SMEM),
                   pl.BlockSpec(memory_space=pltpu.SMEM),
                   pl.BlockSpec(memory_space=pltpu.SMEM)],
        out_shape=[jax.ShapeDtypeStruct((1, 1), jnp.float32),
                   jax.ShapeDtypeStruct((1, 1), jnp.float32),
                   jax.ShapeDtypeStruct((1, 1), jnp.float32)],
    )(xt, bt, a_row, rg, bg, gs, ge)
    return outs[0][0, 0], outs[1][0, 0], outs[2][0, 0]


def _sc_body(bflat, bidx0, bidx1, gs, ge, keep, outr,
             b0i_v, b1i_v, gs_v, ge_v, kp_v, b0_v, b1_v, accr_v, sem):
    cid = lax.axis_index("c")
    sid = lax.axis_index("s")
    wid = sid * 2 + cid
    base = wid * _PW
    pltpu.sync_copy(bidx0.at[pl.ds(base, _PW)], b0i_v)
    pltpu.sync_copy(bidx1.at[pl.ds(base, _PW)], b1i_v)
    pltpu.sync_copy(gs.at[pl.ds(base, _PW)], gs_v)
    pltpu.sync_copy(ge.at[pl.ds(base, _PW)], ge_v)
    pltpu.sync_copy(keep.at[pl.ds(base, _PW)], kp_v)
    pltpu.async_copy(bflat.at[b0i_v], b0_v, sem).wait()
    pltpu.async_copy(bflat.at[b1i_v], b1_v, sem).wait()
    accr = jnp.zeros((16,), jnp.float32)
    for j in range(_PW // 16):
        sl = pl.ds(j * 16, 16)
        k = kp_v[sl]
        for bv, gv in ((b0_v[sl], gs_v[sl]), (b1_v[sl], ge_v[sl])):
            pred = jnp.where(gv != 0.0, bv, jnp.zeros((16,), jnp.float32))
            d = pred - gv
            ad = jnp.abs(d)
            sl1 = jnp.where(ad < 1.0, 0.5 * d * d, ad - 0.5)
            accr = accr + k * sl1
    accr_v[...] = accr
    pltpu.sync_copy(accr_v, outr.at[wid])


def _sc_regression(bboxes, bidx0, gt_bboxes):
    rows = gt_bboxes.shape[0] * gt_bboxes.shape[1]
    pad = _SC_ITEMS - rows
    bidx0_p = jnp.pad(bidx0.reshape(-1)[:rows], (0, pad))
    bidx1_p = bidx0_p + 1
    gs = jnp.pad(gt_bboxes[..., 0].reshape(-1), (0, pad))
    ge = jnp.pad(gt_bboxes[..., 1].reshape(-1), (0, pad))
    keep = jnp.pad(jnp.ones((rows,), jnp.float32), (0, pad))
    mesh = plsc.VectorSubcoreMesh(core_axis_name="c", subcore_axis_name="s")
    outr = pl.kernel(
        _sc_body,
        out_type=jax.ShapeDtypeStruct((32, 16), jnp.float32),
        mesh=mesh,
        scratch_types=[
            pltpu.VMEM((_PW,), jnp.int32),
            pltpu.VMEM((_PW,), jnp.int32),
            pltpu.VMEM((_PW,), jnp.float32),
            pltpu.VMEM((_PW,), jnp.float32),
            pltpu.VMEM((_PW,), jnp.float32),
            pltpu.VMEM((_PW,), jnp.float32),
            pltpu.VMEM((_PW,), jnp.float32),
            pltpu.VMEM((16,), jnp.float32),
            pltpu.SemaphoreType.DMA,
        ],
    )(bboxes.reshape(-1), bidx0_p, bidx1_p, gs, ge, keep)
    return jnp.sum(outr)


def kernel(scores, bboxes, gt_classes, gt_bboxes, anchors):
    rows = gt_classes.size
    besti, bidx0 = _best_anchor_indices(anchors, gt_bboxes)
    a_row = besti.reshape(-1)[:rows].reshape(1, rows)
    s0, corr_c, reg = _dense_loss(scores, bboxes, gt_classes, gt_bboxes,
                                  a_row)
    return s0 - corr_c + reg


# speedup vs baseline: 6.7235x; 1.0011x over previous
import functools

import jax
import jax.numpy as jnp
from jax import lax
from jax.experimental import pallas as pl
from jax.experimental.pallas import tpu as pltpu

_EPS = 1e-8
_CA = 4096
_RP = 800


def _argmax_body(a_s_ref, a_e_ref, b_s_ref, b_e_ref, rowbase_ref,
                 bestk_ref, besti_ref, bidx0_ref):
    i = pl.program_id(0)

    @pl.when(i == 0)
    def _init():
        bestk_ref[...] = jnp.zeros(bestk_ref.shape, jnp.int32)

    a_s = a_s_ref[...]
    a_e = a_e_ref[...]
    b_s = b_s_ref[...]
    b_e = b_e_ref[...]
    inter = jnp.maximum(jnp.minimum(a_e, b_e) - jnp.maximum(a_s, b_s), 0.0)
    denom = (a_e - a_s) + (b_e - b_s + _EPS) - inter
    iou = inter / denom
    bits = lax.bitcast_convert_type(iou, jnp.int32)
    lowb = (jnp.int32(0x7FFF)
            - lax.broadcasted_iota(jnp.int32, (1, _CA), 1)
            - jnp.int32(i * _CA))
    key = (bits & jnp.int32(0x7FFF8000)) | lowb
    mx = jnp.max(key, axis=1, keepdims=True)
    bestk_ref[...] = jnp.maximum(bestk_ref[...], mx)

    @pl.when(i == pl.num_programs(0) - 1)
    def _emit():
        besti = jnp.int32(0x7FFF) - (bestk_ref[...] & jnp.int32(0x7FFF))
        besti_ref[...] = besti
        bidx0_ref[...] = (rowbase_ref[...] + besti) * 2


def _best_anchor_indices(anchors, gt_bboxes):
    b, g = gt_bboxes.shape[0], gt_bboxes.shape[1]
    a = anchors.shape[0]
    a_pad = ((a + _CA - 1) // _CA) * _CA
    rows = b * g
    pad_a = a_pad - a
    a_s = jnp.pad(anchors[:, 0], (0, pad_a), constant_values=2.0).reshape(1, a_pad)
    a_e = jnp.pad(anchors[:, 1], (0, pad_a), constant_values=2.0).reshape(1, a_pad)
    pad_r = _RP - rows
    b_s = jnp.pad(gt_bboxes[..., 0].reshape(-1), (0, pad_r),
                  constant_values=1.0).reshape(_RP, 1)
    b_e = jnp.pad(gt_bboxes[..., 1].reshape(-1), (0, pad_r),
                  constant_values=0.0).reshape(_RP, 1)
    rowbase = jnp.pad(jnp.repeat(jnp.arange(b, dtype=jnp.int32) * a, g),
                      (0, pad_r)).reshape(_RP, 1)

    grid = a_pad // _CA
    row_spec = pl.BlockSpec((_RP, 1), lambda i: (0, 0))
    outs = pl.pallas_call(
        _argmax_body,
        grid=(grid,),
        in_specs=[
            pl.BlockSpec((1, _CA), lambda i: (0, i)),
            pl.BlockSpec((1, _CA), lambda i: (0, i)),
            row_spec, row_spec, row_spec,
        ],
        out_specs=[row_spec, row_spec, row_spec],
        out_shape=[
            jax.ShapeDtypeStruct((_RP, 1), jnp.int32),
            jax.ShapeDtypeStruct((_RP, 1), jnp.int32),
            jax.ShapeDtypeStruct((_RP, 1), jnp.int32),
        ],
    )(a_s, a_e, b_s, b_e, rowbase)
    return outs[1], outs[2]


def _dense_body(xt_ref, bt_ref, ag_ref, rg_ref, bg_ref, gs_ref, ge_ref,
                s0_ref, corr_ref, reg_ref, num_anchors):
    x = xt_ref[...].reshape(xt_ref.shape[0] * xt_ref.shape[1],
                            xt_ref.shape[2])
    bt = bt_ref[...]
    nb = bt.shape[0]
    ag = ag_ref[...]
    rg = rg_ref[...]
    bg = bg_ref[...]
    nrows = ag.shape[1]
    chunk = 1024
    y = jnp.zeros((x.shape[0], nrows), jnp.float32)
    v0 = jnp.zeros((nb, nrows), jnp.float32)
    v1 = jnp.zeros((nb, nrows), jnp.float32)
    ssum = jnp.float32(0.0)
    for s in range(0, num_anchors, chunk):
        n = min(chunk, num_anchors - s)
        xc = x[:, s:s + n]
        ssum += jnp.sum(jnp.maximum(xc, 0.0)
                        + jnp.log1p(jnp.exp(-jnp.abs(xc))))
        ioc = lax.broadcasted_iota(jnp.int32, (n, nrows), 0) + s
        h = jnp.where(ioc == ag, 1.0, 0.0).astype(jnp.bfloat16)
        dims = (((1,), (0,)), ((), ()))
        y = y + lax.dot_general(xc.astype(jnp.bfloat16), h, dims,
                                preferred_element_type=jnp.float32)
        v0 = v0 + lax.dot_general(bt[:, 0, s:s + n].astype(jnp.bfloat16), h,
                                  dims, preferred_element_type=jnp.float32)
        v1 = v1 + lax.dot_general(bt[:, 1, s:s + n].astype(jnp.bfloat16), h,
                                  dims, preferred_element_type=jnp.float32)
    rio = lax.broadcasted_iota(jnp.int32, (x.shape[0], nrows), 0)
    rmask = jnp.where(rio == rg, 1.0, 0.0)
    s0_ref[0, 0] = ssum
    corr_ref[0, 0] = jnp.sum(y * rmask)
    bio = lax.broadcasted_iota(jnp.int32, (nb, nrows), 0)
    bmask = jnp.where(bio == bg, 1.0, 0.0)
    p0 = jnp.sum(v0 * bmask, axis=0, keepdims=True)
    p1 = jnp.sum(v1 * bmask, axis=0, keepdims=True)
    reg = jnp.float32(0.0)
    for pv, gv in ((p0, gs_ref[...]), (p1, ge_ref[...])):
        pred = jnp.where(gv != 0.0, pv, 0.0)
        d = pred - gv
        ad = jnp.abs(d)
        reg += jnp.sum(jnp.where(ad < 1.0, 0.5 * d * d, ad - 0.5))
    reg_ref[0, 0] = reg


def _dense_loss(scores, bboxes, gt_classes, gt_bboxes, a_row):
    b, a, c = scores.shape
    xt = jnp.transpose(scores, (2, 0, 1))
    bt = jnp.transpose(bboxes, (0, 2, 1))
    rows = b * gt_classes.shape[1]
    rg = (gt_classes.astype(jnp.int32) * b
          + jnp.arange(b, dtype=jnp.int32)[:, None]).reshape(1, rows)
    bg = jnp.broadcast_to(jnp.arange(b, dtype=jnp.int32)[:, None],
                          gt_classes.shape).reshape(1, rows)
    gs = gt_bboxes[..., 0].reshape(1, rows)
    ge = gt_bboxes[..., 1].reshape(1, rows)
    row_spec = pl.BlockSpec((1, rows), lambda: (0, 0))
    outs = pl.pallas_call(
        functools.partial(_dense_body, num_anchors=a),
        in_specs=[
            pl.BlockSpec((c, b, a), lambda: (0, 0, 0)),
            pl.BlockSpec((b, 2, a), lambda: (0, 0, 0)),
            row_spec, row_spec, row_spec, row_spec, row_spec,
        ],
        out_specs=[pl.BlockSpec(memory_space=pltpu.SMEM),
                   pl.BlockSpec(memory_space=pltpu.SMEM),
                   pl.BlockSpec(memory_space=pltpu.SMEM)],
        out_shape=[jax.ShapeDtypeStruct((1, 1), jnp.float32),
                   jax.ShapeDtypeStruct((1, 1), jnp.float32),
                   jax.ShapeDtypeStruct((1, 1), jnp.float32)],
    )(xt, bt, a_row, rg, bg, gs, ge)
    return outs[0][0, 0], outs[1][0, 0], outs[2][0, 0]


def kernel(scores, bboxes, gt_classes, gt_bboxes, anchors):
    rows = gt_classes.size
    besti, bidx0 = _best_anchor_indices(anchors, gt_bboxes)
    a_row = besti.reshape(-1)[:rows].reshape(1, rows)
    s0, corr_c, reg = _dense_loss(scores, bboxes, gt_classes, gt_bboxes,
                                  a_row)
    return s0 - corr_c + reg
